# baseline (speedup 1.0000x reference)
import functools

import jax
import jax.numpy as jnp
import numpy as np
from jax import lax
from jax.experimental import pallas as pl
from jax.experimental.pallas import tpu as pltpu
from jax.experimental.pallas import tpu_sc as plsc

E = 8
D = 768
FF = 768
ALPHA = 1.702
LIMIT = 7.0
T = 2048
BK = 256
NB = 2 * T // BK + E
NPAD = NB * BK
NW = 32

_HI = jax.lax.Precision.HIGHEST

_SEL_NP = np.zeros((2 * FF, FF), dtype=np.float32)
_SEL_NP[::2, :] = np.eye(FF, dtype=np.float32)
_SEL = _SEL_NP.astype(jnp.bfloat16)

_L512 = np.tril(np.ones((512, 512), dtype=np.float32))
_U8 = (np.arange(E)[:, None] < np.arange(E)[None, :]).astype(np.float32)


def _plan_body(hs_ref, rwt_ref, rb_ref, l512_ref, u8_ref,
               scores_ref, gidx_ref, wrow_ref, pos1_ref, pos2_ref, bexp_ref):
    x = hs_ref[...]
    logits = jnp.dot(x, rwt_ref[...], preferred_element_type=jnp.float32)
    logits = logits + rb_ref[...]
    col = jax.lax.broadcasted_iota(jnp.int32, logits.shape, 1)
    m1 = jnp.max(logits, axis=1, keepdims=True)
    a1 = jnp.min(jnp.where(logits == m1, col, E), axis=1, keepdims=True)
    rest = jnp.where(col == a1, -jnp.inf, logits)
    m2 = jnp.max(rest, axis=1, keepdims=True)
    a2 = jnp.min(jnp.where(rest == m2, col, E), axis=1, keepdims=True)
    p1 = 1.0 / (1.0 + jnp.exp(m2 - m1))
    p2 = 1.0 - p1
    scores_ref[...] = jnp.where(col == a1, p1, jnp.where(col == a2, p2, 0.0))

    keys = jnp.concatenate([a1, a2], axis=0)
    probs = jnp.concatenate([p1, p2], axis=0)
    tokf = (jax.lax.broadcasted_iota(jnp.int32, (2 * T, 1), 0) % T
            ).astype(jnp.float32)

    erow = jax.lax.broadcasted_iota(jnp.int32, (2 * T, E), 1)
    onehot = (keys == erow).astype(jnp.float32)
    counts = jnp.sum(onehot, axis=0, keepdims=True)
    cnt_pad = jnp.floor((counts + (BK - 1)) * (1.0 / BK)) * BK
    off = jax.lax.dot_general(cnt_pad, u8_ref[...], (((1,), (0,)), ((), ())),
                              precision=_HI,
                              preferred_element_type=jnp.float32)

    l512 = l512_ref[...]
    run = jnp.zeros((1, E), jnp.float32)
    cum_chunks = []
    for c in range(2 * T // 512):
        blk = onehot[c * 512:(c + 1) * 512, :]
        cumblk = jax.lax.dot_general(l512, blk, (((1,), (0,)), ((), ())),
                                     precision=_HI,
                                     preferred_element_type=jnp.float32)
        cum_chunks.append(cumblk + run)
        run = run + jnp.sum(blk, axis=0, keepdims=True)
    cum = jnp.concatenate(cum_chunks, axis=0)

    rank = jnp.sum(onehot * cum, axis=1, keepdims=True)
    offg = jnp.sum(onehot * off, axis=1, keepdims=True)
    pos = offg + rank - 1.0
    pos1_ref[...] = pos[:T].astype(jnp.int32)
    pos2_ref[...] = pos[T:].astype(jnp.int32)

    base_iota = jax.lax.broadcasted_iota(jnp.int32, (2 * T, 512), 1)

    def _scatter_chunk(r, carry):
        rcol = (base_iota + r * 512).astype(jnp.float32)
        m = pos == rcol
        gidx_ref[r, :] = jnp.max(jnp.where(m, tokf, 0.0), axis=0).astype(jnp.int32)
        wrow_ref[r, :] = jnp.max(jnp.where(m, probs, 0.0), axis=0)
        return carry

    jax.lax.fori_loop(0, NPAD // 512, _scatter_chunk, 0)

    offb = jnp.broadcast_to(off, (128, E))
    bio = (jax.lax.broadcasted_iota(jnp.int32, (128, E), 0) * BK
           ).astype(jnp.float32)
    bexp_ref[...] = (jnp.sum((offb <= bio).astype(jnp.int32), axis=1,
                             keepdims=True) - 1)


def _plan(hs, rwt, rb):
    return pl.pallas_call(
        _plan_body,
        grid=(1,),
        in_specs=[
            pl.BlockSpec((T, D), lambda i: (0, 0)),
            pl.BlockSpec((D, E), lambda i: (0, 0)),
            pl.BlockSpec((1, E), lambda i: (0, 0)),
            pl.BlockSpec((512, 512), lambda i: (0, 0)),
            pl.BlockSpec((E, E), lambda i: (0, 0)),
        ],
        out_specs=[
            pl.BlockSpec((T, E), lambda i: (0, 0)),
            pl.BlockSpec((NPAD // 512, 512), lambda i: (0, 0)),
            pl.BlockSpec((NPAD // 512, 512), lambda i: (0, 0)),
            pl.BlockSpec((T, 1), lambda i: (0, 0)),
            pl.BlockSpec((T, 1), lambda i: (0, 0)),
            pl.BlockSpec((128, 1), lambda i: (0, 0)),
        ],
        out_shape=[
            jax.ShapeDtypeStruct((T, E), jnp.float32),
            jax.ShapeDtypeStruct((NPAD // 512, 512), jnp.int32),
            jax.ShapeDtypeStruct((NPAD // 512, 512), jnp.float32),
            jax.ShapeDtypeStruct((T, 1), jnp.int32),
            jax.ShapeDtypeStruct((T, 1), jnp.int32),
            jax.ShapeDtypeStruct((128, 1), jnp.int32),
        ],
    )(hs, rwt, rb, _L512, _U8)


def _sc_gather(table, idx):
    rows_w = NPAD // NW
    half = rows_w // 2
    mesh = plsc.VectorSubcoreMesh(core_axis_name="c", subcore_axis_name="s")

    @functools.partial(
        pl.kernel, mesh=mesh,
        out_type=jax.ShapeDtypeStruct((NPAD, D), jnp.float32),
        scratch_types=[
            pltpu.VMEM((rows_w,), jnp.int32),
            pltpu.VMEM((half, D), jnp.float32),
            pltpu.SemaphoreType.DMA,
        ],
    )
    def k(tbl_hbm, idx_hbm, out_hbm, idx_v, buf, sem):
        wid = lax.axis_index("s") * 2 + lax.axis_index("c")
        base = wid * rows_w
        pltpu.sync_copy(idx_hbm.at[pl.ds(base, rows_w)], idx_v)
        for j in range(2):
            pltpu.async_copy(
                tbl_hbm.at[idx_v.at[pl.ds(j * half, half)]], buf, sem).wait()
            pltpu.sync_copy(buf, out_hbm.at[pl.ds(base + j * half, half)])

    return k(table, idx)


def _sc_unsort(y, pos1, pos2):
    rows_w = T // NW
    mesh = plsc.VectorSubcoreMesh(core_axis_name="c", subcore_axis_name="s")

    @functools.partial(
        pl.kernel, mesh=mesh,
        out_type=(jax.ShapeDtypeStruct((T, D), jnp.float32),
                  jax.ShapeDtypeStruct((T, D), jnp.float32)),
        scratch_types=[
            pltpu.VMEM((rows_w,), jnp.int32),
            pltpu.VMEM((rows_w, D), jnp.float32),
            pltpu.SemaphoreType.DMA,
        ],
    )
    def k(y_hbm, p1_hbm, p2_hbm, y1_hbm, y2_hbm, idx_v, buf, sem):
        wid = lax.axis_index("s") * 2 + lax.axis_index("c")
        base = wid * rows_w
        pltpu.sync_copy(p1_hbm.at[pl.ds(base, rows_w)], idx_v)
        pltpu.async_copy(y_hbm.at[idx_v], buf, sem).wait()
        pltpu.sync_copy(buf, y1_hbm.at[pl.ds(base, rows_w)])
        pltpu.sync_copy(p2_hbm.at[pl.ds(base, rows_w)], idx_v)
        pltpu.async_copy(y_hbm.at[idx_v], buf, sem).wait()
        pltpu.sync_copy(buf, y2_hbm.at[pl.ds(base, rows_w)])

    return k(y, pos1, pos2)


def _ffn_body(bexp_ref, xs_ref, wr_ref, guw_ref, gub_ref, dw_ref, db_ref,
              sel_ref, y_ref):
    x = xs_ref[...]
    xb = x.astype(jnp.bfloat16)
    guw = guw_ref[0].astype(jnp.bfloat16)
    gu = jnp.dot(xb, guw, preferred_element_type=jnp.float32) + gub_ref[0]
    gu_r = pltpu.roll(gu, 2 * FF - 1, 1)
    g = jnp.minimum(gu, LIMIT)
    u = jnp.clip(gu_r, -LIMIT, LIMIT)
    glu = g / (1.0 + jnp.exp(-ALPHA * g))
    act2 = ((u + 1.0) * glu).astype(jnp.bfloat16)
    act = jnp.dot(act2, sel_ref[...], preferred_element_type=jnp.float32)
    act = act.astype(jnp.bfloat16)
    dw = dw_ref[0].astype(jnp.bfloat16)
    contrib = jnp.dot(act, dw, preferred_element_type=jnp.float32)
    y_ref[...] = wr_ref[...] * (contrib + db_ref[0])


def _ffn(bexp, xs, wrow, guw, gub, dw, db, sel):
    grid_spec = pltpu.PrefetchScalarGridSpec(
        num_scalar_prefetch=1,
        grid=(NB,),
        in_specs=[
            pl.BlockSpec((BK, D), lambda b, p: (b, 0)),
            pl.BlockSpec((BK, 1), lambda b, p: (b, 0)),
            pl.BlockSpec((1, D, 2 * FF), lambda b, p: (p[b], 0, 0)),
            pl.BlockSpec((1, 1, 2 * FF), lambda b, p: (p[b], 0, 0)),
            pl.BlockSpec((1, FF, D), lambda b, p: (p[b], 0, 0)),
            pl.BlockSpec((1, 1, D), lambda b, p: (p[b], 0, 0)),
            pl.BlockSpec((2 * FF, FF), lambda b, p: (0, 0)),
        ],
        out_specs=pl.BlockSpec((BK, D), lambda b, p: (b, 0)),
    )
    return pl.pallas_call(
        _ffn_body,
        grid_spec=grid_spec,
        out_shape=jax.ShapeDtypeStruct((NPAD, D), jnp.float32),
    )(bexp, xs, wrow, guw, gub, dw, db, sel)


def _add_body(a_ref, b_ref, o_ref):
    o_ref[...] = a_ref[...] + b_ref[...]


def _add(a, b):
    return pl.pallas_call(
        _add_body,
        grid=(4,),
        in_specs=[pl.BlockSpec((T // 4, D), lambda i: (i, 0)),
                  pl.BlockSpec((T // 4, D), lambda i: (i, 0))],
        out_specs=pl.BlockSpec((T // 4, D), lambda i: (i, 0)),
        out_shape=jax.ShapeDtypeStruct((T, D), jnp.float32),
    )(a, b)


def kernel(hidden_states, router_weight, router_bias, gate_up_proj,
           gate_up_proj_bias, down_proj, down_proj_bias):
    bsz, seq, d = hidden_states.shape
    hs = hidden_states.reshape(T, d)
    rwt = router_weight.T
    rb = router_bias.reshape(1, E)
    gub = gate_up_proj_bias.reshape(E, 1, 2 * FF)
    db = down_proj_bias.reshape(E, 1, D)

    scores, gidx2, wrow2, pos1, pos2, bexp2 = _plan(hs, rwt, rb)
    gidx = gidx2.reshape(NPAD)
    wrow = wrow2.reshape(NPAD, 1)
    bexp = bexp2.reshape(128)[:NB]
    p1v = pos1.reshape(T)
    p2v = pos2.reshape(T)

    xs = _sc_gather(hs, gidx)
    y = _ffn(bexp, xs, wrow, gate_up_proj, gub, down_proj, db, _SEL)
    y1, y2 = _sc_unsort(y, p1v, p2v)
    out = _add(y1, y2)

    return out.reshape(bsz, seq, d), scores


# speedup vs baseline: 1.7909x; 1.7909x over previous
import functools

import jax
import jax.numpy as jnp
import numpy as np
from jax.experimental import pallas as pl
from jax.experimental.pallas import tpu as pltpu

E = 8
D = 768
FF = 768
ALPHA = 1.702
LIMIT = 7.0

_SEL_NP = np.zeros((2 * FF, FF), dtype=np.float32)
_SEL_NP[::2, :] = np.eye(FF, dtype=np.float32)
_SEL = _SEL_NP.astype(jnp.bfloat16)


def _moe_body(hs_ref, rwt_ref, rb_ref, guw_ref, gub_ref, dw_ref, db_ref,
              sel_ref, out_ref, scores_ref, *, bt):
    e = pl.program_id(0)
    t = pl.program_id(1)
    x = hs_ref[...]

    @pl.when(e == 0)
    def _router():
        logits = jnp.dot(x, rwt_ref[...], preferred_element_type=jnp.float32)
        logits = logits + rb_ref[...]
        col = jax.lax.broadcasted_iota(jnp.int32, logits.shape, 1)
        m1 = jnp.max(logits, axis=1, keepdims=True)
        a1 = jnp.min(jnp.where(logits == m1, col, E), axis=1, keepdims=True)
        rest = jnp.where(col == a1, -jnp.inf, logits)
        m2 = jnp.max(rest, axis=1, keepdims=True)
        a2 = jnp.min(jnp.where(rest == m2, col, E), axis=1, keepdims=True)
        p1 = 1.0 / (1.0 + jnp.exp(m2 - m1))
        p2 = 1.0 - p1
        scores = jnp.where(col == a1, p1, jnp.where(col == a2, p2, 0.0))
        scores_ref[pl.ds(t * bt, bt), :] = scores

    scores_blk = scores_ref[pl.ds(t * bt, bt), :]
    col = jax.lax.broadcasted_iota(jnp.int32, scores_blk.shape, 1)
    w = jnp.sum(jnp.where(col == e, scores_blk, 0.0), axis=1, keepdims=True)

    xb = x.astype(jnp.bfloat16)
    guw = guw_ref[0].astype(jnp.bfloat16)
    gu = jnp.dot(xb, guw, preferred_element_type=jnp.float32) + gub_ref[0]
    gu_r = pltpu.roll(gu, 2 * FF - 1, 1)
    g = jnp.minimum(gu, LIMIT)
    u = jnp.clip(gu_r, -LIMIT, LIMIT)
    glu = g / (1.0 + jnp.exp(-ALPHA * g))
    act2 = ((u + 1.0) * glu).astype(jnp.bfloat16)
    act = jnp.dot(act2, sel_ref[...], preferred_element_type=jnp.float32)
    act = act.astype(jnp.bfloat16)
    dw = dw_ref[0].astype(jnp.bfloat16)
    contrib = jnp.dot(act, dw, preferred_element_type=jnp.float32)
    contrib = w * (contrib + db_ref[0])

    sl = pl.ds(t * bt, bt)

    @pl.when(e == 0)
    def _init():
        out_ref[sl, :] = contrib

    @pl.when(e != 0)
    def _acc():
        out_ref[sl, :] = out_ref[sl, :] + contrib


def kernel(hidden_states, router_weight, router_bias, gate_up_proj,
           gate_up_proj_bias, down_proj, down_proj_bias):
    bsz, seq, d = hidden_states.shape
    T = bsz * seq
    hs = hidden_states.reshape(T, d)
    BT = 1024
    NT = T // BT

    rwt = router_weight.T
    rb = router_bias.reshape(1, E)
    gub = gate_up_proj_bias.reshape(E, 1, 2 * FF)
    db = down_proj_bias.reshape(E, 1, D)
    sel = _SEL

    grid = (E, NT)
    out, scores = pl.pallas_call(
        functools.partial(_moe_body, bt=BT),
        grid=grid,
        in_specs=[
            pl.BlockSpec((BT, D), lambda e, t: (t, 0)),
            pl.BlockSpec((D, E), lambda e, t: (0, 0)),
            pl.BlockSpec((1, E), lambda e, t: (0, 0)),
            pl.BlockSpec((1, D, 2 * FF), lambda e, t: (e, 0, 0)),
            pl.BlockSpec((1, 1, 2 * FF), lambda e, t: (e, 0, 0)),
            pl.BlockSpec((1, FF, D), lambda e, t: (e, 0, 0)),
            pl.BlockSpec((1, 1, D), lambda e, t: (e, 0, 0)),
            pl.BlockSpec((2 * FF, FF), lambda e, t: (0, 0)),
        ],
        out_specs=[
            pl.BlockSpec((T, D), lambda e, t: (0, 0)),
            pl.BlockSpec((T, E), lambda e, t: (0, 0)),
        ],
        out_shape=[
            jax.ShapeDtypeStruct((T, D), jnp.float32),
            jax.ShapeDtypeStruct((T, E), jnp.float32),
        ],
        compiler_params=pltpu.CompilerParams(
            dimension_semantics=("arbitrary", "arbitrary"),
        ),
    )(hs, rwt, rb, gate_up_proj, gub, down_proj, db, sel)

    return out.reshape(bsz, seq, d), scores
